# Initial kernel scaffold; baseline (speedup 1.0000x reference)
import functools

import jax
import jax.numpy as jnp
from jax import lax
from jax.experimental import pallas as pl
from jax.experimental.pallas import tpu as pltpu

_K = 1024
_D = 256
_R = 512
_N = 16384


def _vq_block(x_ref, x2_ref, et_ref, e2_ref, emb_ref,
              q_ref, enc_ref, idx_ref, loss_ref):
    i = pl.program_id(0)
    x = x_ref[...]
    m = jnp.dot(x, et_ref[...],
                preferred_element_type=jnp.float32)
    d = (x2_ref[...] + e2_ref[...]) - 2.0 * m
    minv = jnp.min(d, axis=1, keepdims=True)
    iota = lax.broadcasted_iota(jnp.int32, (_R, _K), 1)
    idx = jnp.min(jnp.where(d == minv, iota, _K), axis=1)
    enc = (iota == idx[:, None]).astype(jnp.float32)
    enc_ref[...] = enc
    idx_ref[0, 0, :] = idx
    g = jnp.dot(enc, emb_ref[...],
                preferred_element_type=jnp.float32)
    q_ref[...] = x + (g - x)

    @pl.when(i == 0)
    def _():
        loss_ref[0, 0] = 0.0

    loss_ref[0, 0] += jnp.sum((g - x) ** 2)


def kernel(inputs, embedding):
    input_shape = inputs.shape
    flat = inputs.reshape(-1, _D)
    x2 = jnp.sum(flat ** 2, axis=1, keepdims=True)
    e2 = jnp.sum(embedding ** 2, axis=1)[None, :]
    et = embedding.T

    grid = _N // _R
    q, enc, idx3, loss_acc = pl.pallas_call(
        _vq_block,
        grid=(grid,),
        in_specs=[
            pl.BlockSpec((_R, _D), lambda i: (i, 0)),
            pl.BlockSpec((_R, 1), lambda i: (i, 0)),
            pl.BlockSpec((_D, _K), lambda i: (0, 0)),
            pl.BlockSpec((1, _K), lambda i: (0, 0)),
            pl.BlockSpec((_K, _D), lambda i: (0, 0)),
        ],
        out_specs=[
            pl.BlockSpec((_R, _D), lambda i: (i, 0)),
            pl.BlockSpec((_R, _K), lambda i: (i, 0)),
            pl.BlockSpec((1, 1, _R), lambda i: (i, 0, 0)),
            pl.BlockSpec((1, 1), lambda i: (0, 0)),
        ],
        out_shape=[
            jax.ShapeDtypeStruct((_N, _D), jnp.float32),
            jax.ShapeDtypeStruct((_N, _K), jnp.float32),
            jax.ShapeDtypeStruct((grid, 1, _R), jnp.int32),
            jax.ShapeDtypeStruct((1, 1), jnp.float32),
        ],
        compiler_params=pltpu.CompilerParams(
            dimension_semantics=("arbitrary",),
        ),
    )(flat, x2, et, e2, embedding)

    mean_sq = loss_acc[0, 0] / (_N * _D)
    loss = mean_sq + 0.25 * mean_sq
    quantized = q.reshape(input_shape)
    encoding_indices = idx3.reshape(input_shape[:-1])
    return (quantized, loss, enc, encoding_indices)


# speedup vs baseline: 1.6560x; 1.6560x over previous
import functools

import jax
import jax.numpy as jnp
from jax import lax
from jax.experimental import pallas as pl
from jax.experimental.pallas import tpu as pltpu

_K = 1024
_D = 256
_R = 512
_N = 16384


def _vq_block(x_ref, x2_ref, et_ref, e2_ref, emb_ref,
              q_ref, enc_ref, idx_ref, loss_ref):
    i = pl.program_id(0)
    x = x_ref[...]
    m = jnp.dot(x, et_ref[...],
                preferred_element_type=jnp.float32)
    d = (x2_ref[...] + e2_ref[...]) - 2.0 * m
    minv = jnp.min(d, axis=1, keepdims=True)
    iota = lax.broadcasted_iota(jnp.int32, (_R, _K), 1)
    idx = jnp.min(jnp.where(d == minv, iota, _K), axis=1)
    enc = (iota == idx[:, None]).astype(jnp.float32)
    enc_ref[...] = enc
    idx_ref[0, 0, :] = idx
    g = jnp.dot(enc, emb_ref[...],
                preferred_element_type=jnp.float32)
    q_ref[...] = x + (g - x)

    @pl.when(i == 0)
    def _():
        loss_ref[...] = jnp.zeros((1, 1), jnp.float32)

    loss_ref[...] += jnp.sum((g - x) ** 2).reshape(1, 1)


def kernel(inputs, embedding):
    input_shape = inputs.shape
    flat = inputs.reshape(-1, _D)
    x2 = jnp.sum(flat ** 2, axis=1, keepdims=True)
    e2 = jnp.sum(embedding ** 2, axis=1)[None, :]
    et = embedding.T

    grid = _N // _R
    q, enc, idx3, loss_acc = pl.pallas_call(
        _vq_block,
        grid=(grid,),
        in_specs=[
            pl.BlockSpec((_R, _D), lambda i: (i, 0)),
            pl.BlockSpec((_R, 1), lambda i: (i, 0)),
            pl.BlockSpec((_D, _K), lambda i: (0, 0)),
            pl.BlockSpec((1, _K), lambda i: (0, 0)),
            pl.BlockSpec((_K, _D), lambda i: (0, 0)),
        ],
        out_specs=[
            pl.BlockSpec((_R, _D), lambda i: (i, 0)),
            pl.BlockSpec((_R, _K), lambda i: (i, 0)),
            pl.BlockSpec((1, 1, _R), lambda i: (i, 0, 0)),
            pl.BlockSpec((1, 1), lambda i: (0, 0)),
        ],
        out_shape=[
            jax.ShapeDtypeStruct((_N, _D), jnp.float32),
            jax.ShapeDtypeStruct((_N, _K), jnp.float32),
            jax.ShapeDtypeStruct((grid, 1, _R), jnp.int32),
            jax.ShapeDtypeStruct((1, 1), jnp.float32),
        ],
        compiler_params=pltpu.CompilerParams(
            dimension_semantics=("arbitrary",),
        ),
    )(flat, x2, et, e2, embedding)

    mean_sq = loss_acc[0, 0] / (_N * _D)
    loss = mean_sq + 0.25 * mean_sq
    quantized = q.reshape(input_shape)
    encoding_indices = idx3.reshape(input_shape[:-1])
    return (quantized, loss, enc, encoding_indices)
